# Initial kernel scaffold; baseline (speedup 1.0000x reference)
import jax
import jax.numpy as jnp
from jax.experimental import pallas as pl


def kernel(x, adj, W1, b1, W2, b2):
    raise NotImplementedError("write your pallas kernel here")


# speedup vs baseline: 4.1129x; 4.1129x over previous
import jax
import jax.numpy as jnp
from jax import lax
from jax.experimental import pallas as pl
from jax.experimental.pallas import tpu as pltpu
from jax.experimental.pallas import tpu_sc as plsc

N = 10000
E = 160000
F = 256
H = 256
C = 64

NP = 10240
TRASH = 10000
EP = 163840
NCORE = 2
NTILE = 16
ROWS_PER_TILE = NP // NTILE
A_CHUNK = EP // (NCORE * NTILE)
B_CHUNKS = (EP // NTILE) // 128

_mesh = plsc.VectorSubcoreMesh(
    core_axis_name="c", subcore_axis_name="s",
    num_cores=NCORE, num_subcores=NTILE)


def _deg_body(r_in, c_in, r_out, deg_parts, rbuf, cbuf, degbuf):
  cid = lax.axis_index("c")
  sid = lax.axis_index("s")
  wid = sid * NCORE + cid
  off = wid * A_CHUNK
  pltpu.sync_copy(r_in.at[pl.ds(off, A_CHUNK)], rbuf)
  pltpu.sync_copy(c_in.at[pl.ds(off, A_CHUNK)], cbuf)

  zeros16 = jnp.zeros((16,), jnp.float32)
  def zbody(i, carry):
    degbuf[pl.ds(i * 16, 16)] = zeros16
    return carry
  lax.fori_loop(0, NP // 16, zbody, 0)

  ones16 = jnp.ones((16,), jnp.float32)
  def ebody(i, carry):
    rv = rbuf[pl.ds(i * 16, 16)]
    cv = cbuf[pl.ds(i * 16, 16)]
    r2 = jnp.where(rv != cv, rv, TRASH)
    plsc.addupdate_scatter(degbuf, [r2], ones16)
    rbuf[pl.ds(i * 16, 16)] = r2
    return carry
  lax.fori_loop(0, A_CHUNK // 16, ebody, 0)

  pltpu.sync_copy(rbuf, r_out.at[pl.ds(off, A_CHUNK)])
  pltpu.sync_copy(degbuf, deg_parts.at[wid])


_deg_kernel = pl.kernel(
    _deg_body,
    out_type=(jax.ShapeDtypeStruct((EP,), jnp.int32),
              jax.ShapeDtypeStruct((NCORE * NTILE, NP), jnp.float32)),
    mesh=_mesh,
    scratch_types=(pltpu.VMEM((A_CHUNK,), jnp.int32),
                   pltpu.VMEM((A_CHUNK,), jnp.int32),
                   pltpu.VMEM((NP,), jnp.float32)),
    compiler_params=pltpu.CompilerParams(needs_layout_passes=False),
)


def _spmm_body(t0, t1, cidx, ridx, zrows, o0, o1, cbuf, rbuf, gbuf, acc, sem):
  cid = lax.axis_index("c")
  sid = lax.axis_index("s")
  row0 = sid * ROWS_PER_TILE
  pltpu.sync_copy(zrows, acc.at[pl.ds(row0, ROWS_PER_TILE)])
  pltpu.sync_copy(cidx.at[sid], cbuf)
  pltpu.sync_copy(ridx.at[sid], rbuf)
  plsc.subcore_barrier()

  def run(table):
    def body(j, carry):
      pltpu.async_copy(table.at[cbuf.at[j]], gbuf, sem).wait()
      pltpu.sync_copy(gbuf, acc.at[rbuf.at[j]], add=True)
      return carry
    lax.fori_loop(0, B_CHUNKS, body, 0)

  @pl.when(cid == 0)
  def _():
    run(t0)
  @pl.when(cid == 1)
  def _():
    run(t1)

  plsc.subcore_barrier()

  @pl.when(cid == 0)
  def _():
    pltpu.sync_copy(acc.at[pl.ds(row0, ROWS_PER_TILE)],
                    o0.at[pl.ds(row0, ROWS_PER_TILE)])
  @pl.when(cid == 1)
  def _():
    pltpu.sync_copy(acc.at[pl.ds(row0, ROWS_PER_TILE)],
                    o1.at[pl.ds(row0, ROWS_PER_TILE)])


def _make_spmm(fw):
  return pl.kernel(
      _spmm_body,
      out_type=(jax.ShapeDtypeStruct((NP, fw), jnp.float32),
                jax.ShapeDtypeStruct((NP, fw), jnp.float32)),
      mesh=_mesh,
      scratch_types=(
          pltpu.VMEM((B_CHUNKS, 128), jnp.int32),
          pltpu.VMEM((B_CHUNKS, 128), jnp.int32),
          pltpu.VMEM((128, fw), jnp.float32),
          pltpu.VMEM_SHARED((NP, fw), jnp.float32),
          pltpu.SemaphoreType.DMA,
      ),
      compiler_params=pltpu.CompilerParams(use_tc_tiling_on_sc=False),
  )

_spmm_h = _make_spmm(H // 2)
_spmm_c = _make_spmm(C // 2)


BLK = 512
GRID = NP // BLK


def _mm1_body(x_ref, w_ref, deg_ref, dinv_ref, s0_ref, s1_ref):
  deg = jnp.sum(deg_ref[...], axis=0)
  dinv = jnp.where(deg > 0.0, lax.rsqrt(deg), 0.0)
  dinv_ref[...] = dinv
  s = jnp.dot(x_ref[...], w_ref[...], preferred_element_type=jnp.float32)
  s = s * dinv[:, None]
  s0_ref[...] = s[:, :H // 2]
  s1_ref[...] = s[:, H // 2:]


_mm1 = pl.pallas_call(
    _mm1_body,
    grid=(GRID,),
    in_specs=[pl.BlockSpec((BLK, F), lambda i: (i, 0)),
              pl.BlockSpec((F, H), lambda i: (0, 0)),
              pl.BlockSpec((NCORE * NTILE, BLK), lambda i: (0, i))],
    out_specs=[pl.BlockSpec((BLK,), lambda i: (i,)),
               pl.BlockSpec((BLK, H // 2), lambda i: (i, 0)),
               pl.BlockSpec((BLK, H // 2), lambda i: (i, 0))],
    out_shape=[jax.ShapeDtypeStruct((NP,), jnp.float32),
               jax.ShapeDtypeStruct((NP, H // 2), jnp.float32),
               jax.ShapeDtypeStruct((NP, H // 2), jnp.float32)],
)


def _mm2_body(t0_ref, t1_ref, dinv_ref, b1_ref, w2_ref, s0_ref, s1_ref):
  dv = dinv_ref[...]
  t = jnp.concatenate([t0_ref[...], t1_ref[...]], axis=1)
  h = jnp.maximum(t * dv[:, None] + b1_ref[...], 0.0)
  s2 = jnp.dot(h, w2_ref[...], preferred_element_type=jnp.float32)
  s2 = s2 * dv[:, None]
  s0_ref[...] = s2[:, :C // 2]
  s1_ref[...] = s2[:, C // 2:]


_mm2 = pl.pallas_call(
    _mm2_body,
    grid=(GRID,),
    in_specs=[pl.BlockSpec((BLK, H // 2), lambda i: (i, 0)),
              pl.BlockSpec((BLK, H // 2), lambda i: (i, 0)),
              pl.BlockSpec((BLK,), lambda i: (i,)),
              pl.BlockSpec((1, H), lambda i: (0, 0)),
              pl.BlockSpec((H, C), lambda i: (0, 0))],
    out_specs=[pl.BlockSpec((BLK, C // 2), lambda i: (i, 0)),
               pl.BlockSpec((BLK, C // 2), lambda i: (i, 0))],
    out_shape=[jax.ShapeDtypeStruct((NP, C // 2), jnp.float32),
               jax.ShapeDtypeStruct((NP, C // 2), jnp.float32)],
)


def _mm3_body(t0_ref, t1_ref, dinv_ref, b2_ref, o_ref):
  dv = dinv_ref[...]
  t = jnp.concatenate([t0_ref[...], t1_ref[...]], axis=1)
  o_ref[...] = t * dv[:, None] + b2_ref[...]


_mm3 = pl.pallas_call(
    _mm3_body,
    grid=(GRID,),
    in_specs=[pl.BlockSpec((BLK, C // 2), lambda i: (i, 0)),
              pl.BlockSpec((BLK, C // 2), lambda i: (i, 0)),
              pl.BlockSpec((BLK,), lambda i: (i,)),
              pl.BlockSpec((1, C), lambda i: (0, 0))],
    out_specs=pl.BlockSpec((BLK, C), lambda i: (i, 0)),
    out_shape=jax.ShapeDtypeStruct((NP, C), jnp.float32),
)


def kernel(x, adj, W1, b1, W2, b2):
  r = jnp.concatenate([adj[0], jnp.zeros((EP - E,), jnp.int32)])
  c = jnp.concatenate([adj[1], jnp.zeros((EP - E,), jnp.int32)])
  x_pad = jnp.concatenate([x, jnp.zeros((NP - N, F), x.dtype)])

  r_adj, deg_parts = _deg_kernel(r, c)
  cidx = c.reshape(NTILE, B_CHUNKS, 128)
  ridx = r_adj.reshape(NTILE, B_CHUNKS, 128)

  dinv2d, s1a, s1b = _mm1(x_pad, W1, deg_parts)

  zh = jnp.zeros((ROWS_PER_TILE, H // 2), jnp.float32)
  t1a, t1b = _spmm_h(s1a, s1b, cidx, ridx, zh)

  s2a, s2b = _mm2(t1a, t1b, dinv2d, b1.reshape(1, H), W2)

  zc = jnp.zeros((ROWS_PER_TILE, C // 2), jnp.float32)
  t2a, t2b = _spmm_c(s2a, s2b, cidx, ridx, zc)

  out_pad = _mm3(t2a, t2b, dinv2d, b2.reshape(1, C))
  return out_pad[:N]


# speedup vs baseline: 4.7478x; 1.1544x over previous
import jax
import jax.numpy as jnp
from jax import lax
from jax.experimental import pallas as pl
from jax.experimental.pallas import tpu as pltpu
from jax.experimental.pallas import tpu_sc as plsc

N = 10000
E = 160000
F = 256
H = 256
C = 64

NP = 10240
TRASH = 10000
EP = 163840
NCORE = 2
NTILE = 16
ROWS_PER_TILE = NP // NTILE
A_CHUNK = EP // (NCORE * NTILE)
CH = 64
B_CHUNKS = (EP // NTILE) // CH

_mesh = plsc.VectorSubcoreMesh(
    core_axis_name="c", subcore_axis_name="s",
    num_cores=NCORE, num_subcores=NTILE)


def _deg_body(r_in, c_in, r_out, deg_parts, rbuf, cbuf, degbuf):
  cid = lax.axis_index("c")
  sid = lax.axis_index("s")
  wid = sid * NCORE + cid
  off = wid * A_CHUNK
  pltpu.sync_copy(r_in.at[pl.ds(off, A_CHUNK)], rbuf)
  pltpu.sync_copy(c_in.at[pl.ds(off, A_CHUNK)], cbuf)

  zeros16 = jnp.zeros((16,), jnp.float32)
  def zbody(i, carry):
    degbuf[pl.ds(i * 16, 16)] = zeros16
    return carry
  lax.fori_loop(0, NP // 16, zbody, 0)

  ones16 = jnp.ones((16,), jnp.float32)
  def ebody(i, carry):
    rv = rbuf[pl.ds(i * 16, 16)]
    cv = cbuf[pl.ds(i * 16, 16)]
    r2 = jnp.where(rv != cv, rv, TRASH)
    plsc.addupdate_scatter(degbuf, [r2], ones16)
    rbuf[pl.ds(i * 16, 16)] = r2
    return carry
  lax.fori_loop(0, A_CHUNK // 16, ebody, 0)

  pltpu.sync_copy(rbuf, r_out.at[pl.ds(off, A_CHUNK)])
  pltpu.sync_copy(degbuf, deg_parts.at[wid])


_deg_kernel = pl.kernel(
    _deg_body,
    out_type=(jax.ShapeDtypeStruct((EP,), jnp.int32),
              jax.ShapeDtypeStruct((NCORE * NTILE, NP), jnp.float32)),
    mesh=_mesh,
    scratch_types=(pltpu.VMEM((A_CHUNK,), jnp.int32),
                   pltpu.VMEM((A_CHUNK,), jnp.int32),
                   pltpu.VMEM((NP,), jnp.float32)),
    compiler_params=pltpu.CompilerParams(needs_layout_passes=False),
)


NBUF = 2


def _spmm_body(t0, t1, cidx, ridx, zrows, o0, o1, cbuf, rbuf,
               g0, g1, acc, s0, s1):
  cid = lax.axis_index("c")
  sid = lax.axis_index("s")
  row0 = sid * ROWS_PER_TILE
  gbufs = (g0, g1)
  sems = (s0, s1)
  pltpu.sync_copy(zrows, acc.at[pl.ds(row0, ROWS_PER_TILE)])
  pltpu.sync_copy(cidx.at[sid], cbuf)
  pltpu.sync_copy(ridx.at[sid], rbuf)
  plsc.subcore_barrier()

  def run(table):
    for k in range(NBUF):
      pltpu.async_copy(table.at[cbuf.at[k]], gbufs[k], sems[k])
    def body(i, carry):
      for k in range(NBUF):
        j = i * NBUF + k
        pltpu.make_async_copy(table.at[cbuf.at[j]], gbufs[k], sems[k]).wait()
        pltpu.sync_copy(gbufs[k], acc.at[rbuf.at[j]], add=True)
        @pl.when(j + NBUF < B_CHUNKS)
        def _():
          pltpu.async_copy(table.at[cbuf.at[j + NBUF]], gbufs[k], sems[k])
      return carry
    lax.fori_loop(0, B_CHUNKS // NBUF, body, 0)

  @pl.when(cid == 0)
  def _():
    run(t0)
  @pl.when(cid == 1)
  def _():
    run(t1)

  plsc.subcore_barrier()

  @pl.when(cid == 0)
  def _():
    pltpu.sync_copy(acc.at[pl.ds(row0, ROWS_PER_TILE)],
                    o0.at[pl.ds(row0, ROWS_PER_TILE)])
  @pl.when(cid == 1)
  def _():
    pltpu.sync_copy(acc.at[pl.ds(row0, ROWS_PER_TILE)],
                    o1.at[pl.ds(row0, ROWS_PER_TILE)])


def _make_spmm(fw):
  return pl.kernel(
      _spmm_body,
      out_type=(jax.ShapeDtypeStruct((NP, fw), jnp.float32),
                jax.ShapeDtypeStruct((NP, fw), jnp.float32)),
      mesh=_mesh,
      scratch_types=(
          pltpu.VMEM((B_CHUNKS, CH), jnp.int32),
          pltpu.VMEM((B_CHUNKS, CH), jnp.int32),
          pltpu.VMEM((CH, fw), jnp.float32),
          pltpu.VMEM((CH, fw), jnp.float32),
          pltpu.VMEM_SHARED((NP, fw), jnp.float32),
          pltpu.SemaphoreType.DMA,
          pltpu.SemaphoreType.DMA,
      ),
      compiler_params=pltpu.CompilerParams(use_tc_tiling_on_sc=False),
  )

_spmm_h = _make_spmm(H // 2)
_spmm_c = _make_spmm(C // 2)


BLK = 512
GRID = NP // BLK


def _mm1_body(x_ref, w_ref, deg_ref, dinv_ref, s0_ref, s1_ref):
  deg = jnp.sum(deg_ref[...], axis=0)
  dinv = jnp.where(deg > 0.0, lax.rsqrt(deg), 0.0)
  dinv_ref[...] = dinv
  s = jnp.dot(x_ref[...], w_ref[...], preferred_element_type=jnp.float32)
  s = s * dinv[:, None]
  s0_ref[...] = s[:, :H // 2]
  s1_ref[...] = s[:, H // 2:]


_mm1 = pl.pallas_call(
    _mm1_body,
    grid=(GRID,),
    in_specs=[pl.BlockSpec((BLK, F), lambda i: (i, 0)),
              pl.BlockSpec((F, H), lambda i: (0, 0)),
              pl.BlockSpec((NCORE * NTILE, BLK), lambda i: (0, i))],
    out_specs=[pl.BlockSpec((BLK,), lambda i: (i,)),
               pl.BlockSpec((BLK, H // 2), lambda i: (i, 0)),
               pl.BlockSpec((BLK, H // 2), lambda i: (i, 0))],
    out_shape=[jax.ShapeDtypeStruct((NP,), jnp.float32),
               jax.ShapeDtypeStruct((NP, H // 2), jnp.float32),
               jax.ShapeDtypeStruct((NP, H // 2), jnp.float32)],
)


def _mm2_body(t0_ref, t1_ref, dinv_ref, b1_ref, w2_ref, s0_ref, s1_ref):
  dv = dinv_ref[...]
  t = jnp.concatenate([t0_ref[...], t1_ref[...]], axis=1)
  h = jnp.maximum(t * dv[:, None] + b1_ref[...], 0.0)
  s2 = jnp.dot(h, w2_ref[...], preferred_element_type=jnp.float32)
  s2 = s2 * dv[:, None]
  s0_ref[...] = s2[:, :C // 2]
  s1_ref[...] = s2[:, C // 2:]


_mm2 = pl.pallas_call(
    _mm2_body,
    grid=(GRID,),
    in_specs=[pl.BlockSpec((BLK, H // 2), lambda i: (i, 0)),
              pl.BlockSpec((BLK, H // 2), lambda i: (i, 0)),
              pl.BlockSpec((BLK,), lambda i: (i,)),
              pl.BlockSpec((1, H), lambda i: (0, 0)),
              pl.BlockSpec((H, C), lambda i: (0, 0))],
    out_specs=[pl.BlockSpec((BLK, C // 2), lambda i: (i, 0)),
               pl.BlockSpec((BLK, C // 2), lambda i: (i, 0))],
    out_shape=[jax.ShapeDtypeStruct((NP, C // 2), jnp.float32),
               jax.ShapeDtypeStruct((NP, C // 2), jnp.float32)],
)


def _mm3_body(t0_ref, t1_ref, dinv_ref, b2_ref, o_ref):
  dv = dinv_ref[...]
  t = jnp.concatenate([t0_ref[...], t1_ref[...]], axis=1)
  o_ref[...] = t * dv[:, None] + b2_ref[...]


_mm3 = pl.pallas_call(
    _mm3_body,
    grid=(GRID,),
    in_specs=[pl.BlockSpec((BLK, C // 2), lambda i: (i, 0)),
              pl.BlockSpec((BLK, C // 2), lambda i: (i, 0)),
              pl.BlockSpec((BLK,), lambda i: (i,)),
              pl.BlockSpec((1, C), lambda i: (0, 0))],
    out_specs=pl.BlockSpec((BLK, C), lambda i: (i, 0)),
    out_shape=jax.ShapeDtypeStruct((NP, C), jnp.float32),
)


def kernel(x, adj, W1, b1, W2, b2):
  r = jnp.concatenate([adj[0], jnp.zeros((EP - E,), jnp.int32)])
  c = jnp.concatenate([adj[1], jnp.zeros((EP - E,), jnp.int32)])
  x_pad = jnp.concatenate([x, jnp.zeros((NP - N, F), x.dtype)])

  r_adj, deg_parts = _deg_kernel(r, c)
  cidx = c.reshape(NTILE, B_CHUNKS, CH)
  ridx = r_adj.reshape(NTILE, B_CHUNKS, CH)

  dinv2d, s1a, s1b = _mm1(x_pad, W1, deg_parts)

  zh = jnp.zeros((ROWS_PER_TILE, H // 2), jnp.float32)
  t1a, t1b = _spmm_h(s1a, s1b, cidx, ridx, zh)

  s2a, s2b = _mm2(t1a, t1b, dinv2d, b1.reshape(1, H), W2)

  zc = jnp.zeros((ROWS_PER_TILE, C // 2), jnp.float32)
  t2a, t2b = _spmm_c(s2a, s2b, cidx, ridx, zc)

  out_pad = _mm3(t2a, t2b, dinv2d, b2.reshape(1, C))
  return out_pad[:N]


# speedup vs baseline: 4.9838x; 1.0497x over previous
import jax
import jax.numpy as jnp
from jax import lax
from jax.experimental import pallas as pl
from jax.experimental.pallas import tpu as pltpu
from jax.experimental.pallas import tpu_sc as plsc

N = 10000
E = 160000
F = 256
H = 256
C = 64

NP = 10240
TRASH = 10000
EP = 163840
NCORE = 2
NTILE = 16
ROWS_PER_TILE = NP // NTILE
A_CHUNK = EP // (NCORE * NTILE)


_mesh = plsc.VectorSubcoreMesh(
    core_axis_name="c", subcore_axis_name="s",
    num_cores=NCORE, num_subcores=NTILE)


def _deg_body(r_in, c_in, r_out, deg_parts, rbuf, cbuf, degbuf):
  cid = lax.axis_index("c")
  sid = lax.axis_index("s")
  wid = sid * NCORE + cid
  off = wid * A_CHUNK
  pltpu.sync_copy(r_in.at[pl.ds(off, A_CHUNK)], rbuf)
  pltpu.sync_copy(c_in.at[pl.ds(off, A_CHUNK)], cbuf)

  zeros16 = jnp.zeros((16,), jnp.float32)
  def zbody(i, carry):
    degbuf[pl.ds(i * 16, 16)] = zeros16
    return carry
  lax.fori_loop(0, NP // 16, zbody, 0)

  ones16 = jnp.ones((16,), jnp.float32)
  def ebody(i, carry):
    rv = rbuf[pl.ds(i * 16, 16)]
    cv = cbuf[pl.ds(i * 16, 16)]
    r2 = jnp.where(rv != cv, rv, TRASH)
    plsc.addupdate_scatter(degbuf, [r2], ones16)
    rbuf[pl.ds(i * 16, 16)] = r2
    return carry
  lax.fori_loop(0, A_CHUNK // 16, ebody, 0)

  pltpu.sync_copy(rbuf, r_out.at[pl.ds(off, A_CHUNK)])
  pltpu.sync_copy(degbuf, deg_parts.at[wid])


_deg_kernel = pl.kernel(
    _deg_body,
    out_type=(jax.ShapeDtypeStruct((EP,), jnp.int32),
              jax.ShapeDtypeStruct((NCORE * NTILE, NP), jnp.float32)),
    mesh=_mesh,
    scratch_types=(pltpu.VMEM((A_CHUNK,), jnp.int32),
                   pltpu.VMEM((A_CHUNK,), jnp.int32),
                   pltpu.VMEM((NP,), jnp.float32)),
    compiler_params=pltpu.CompilerParams(needs_layout_passes=False),
)


def _make_spmm_body(ch, nbuf, nchunks):
  def _spmm_body(t0, t1, cidx, ridx, zrows, o0, o1, cbuf, rbuf, *rest):
    gbufs = rest[:nbuf]
    acc = rest[nbuf]
    sems = rest[nbuf + 1:]
    cid = lax.axis_index("c")
    sid = lax.axis_index("s")
    row0 = sid * ROWS_PER_TILE
    pltpu.sync_copy(zrows, acc.at[pl.ds(row0, ROWS_PER_TILE)])
    pltpu.sync_copy(cidx.at[sid], cbuf)
    pltpu.sync_copy(ridx.at[sid], rbuf)
    plsc.subcore_barrier()

    def run(table):
      for k in range(nbuf):
        pltpu.async_copy(table.at[cbuf.at[k]], gbufs[k], sems[k])
      def body(i, carry):
        for k in range(nbuf):
          j = i * nbuf + k
          pltpu.make_async_copy(table.at[cbuf.at[j]], gbufs[k], sems[k]).wait()
          pltpu.sync_copy(gbufs[k], acc.at[rbuf.at[j]], add=True)
          @pl.when(j + nbuf < nchunks)
          def _():
            pltpu.async_copy(table.at[cbuf.at[j + nbuf]], gbufs[k], sems[k])
        return carry
      lax.fori_loop(0, nchunks // nbuf, body, 0)

    @pl.when(cid == 0)
    def _():
      run(t0)
    @pl.when(cid == 1)
    def _():
      run(t1)

    plsc.subcore_barrier()

    @pl.when(cid == 0)
    def _():
      pltpu.sync_copy(acc.at[pl.ds(row0, ROWS_PER_TILE)],
                      o0.at[pl.ds(row0, ROWS_PER_TILE)])
    @pl.when(cid == 1)
    def _():
      pltpu.sync_copy(acc.at[pl.ds(row0, ROWS_PER_TILE)],
                      o1.at[pl.ds(row0, ROWS_PER_TILE)])
  return _spmm_body


def _make_spmm(fw, ch, nbuf):
  nchunks = (EP // NTILE) // ch
  return pl.kernel(
      _make_spmm_body(ch, nbuf, nchunks),
      out_type=(jax.ShapeDtypeStruct((NP, fw), jnp.float32),
                jax.ShapeDtypeStruct((NP, fw), jnp.float32)),
      mesh=_mesh,
      scratch_types=(
          pltpu.VMEM((nchunks, ch), jnp.int32),
          pltpu.VMEM((nchunks, ch), jnp.int32),
          *[pltpu.VMEM((ch, fw), jnp.float32) for _ in range(nbuf)],
          pltpu.VMEM_SHARED((NP, fw), jnp.float32),
          *[pltpu.SemaphoreType.DMA for _ in range(nbuf)],
      ),
      compiler_params=pltpu.CompilerParams(use_tc_tiling_on_sc=False),
  )


CH_H = 64
CH_C = 128
_spmm_h = _make_spmm(H // 2, CH_H, 2)
_spmm_c = _make_spmm(C // 2, CH_C, 4)


BLK = 512
GRID = NP // BLK


def _mm1_body(x_ref, w_ref, deg_ref, dinv_ref, s0_ref, s1_ref):
  deg = jnp.sum(deg_ref[...], axis=0)
  dinv = jnp.where(deg > 0.0, lax.rsqrt(deg), 0.0)
  dinv_ref[...] = dinv
  s = jnp.dot(x_ref[...], w_ref[...], preferred_element_type=jnp.float32)
  s = s * dinv[:, None]
  s0_ref[...] = s[:, :H // 2]
  s1_ref[...] = s[:, H // 2:]


_mm1 = pl.pallas_call(
    _mm1_body,
    grid=(GRID,),
    in_specs=[pl.BlockSpec((BLK, F), lambda i: (i, 0)),
              pl.BlockSpec((F, H), lambda i: (0, 0)),
              pl.BlockSpec((NCORE * NTILE, BLK), lambda i: (0, i))],
    out_specs=[pl.BlockSpec((BLK,), lambda i: (i,)),
               pl.BlockSpec((BLK, H // 2), lambda i: (i, 0)),
               pl.BlockSpec((BLK, H // 2), lambda i: (i, 0))],
    out_shape=[jax.ShapeDtypeStruct((NP,), jnp.float32),
               jax.ShapeDtypeStruct((NP, H // 2), jnp.float32),
               jax.ShapeDtypeStruct((NP, H // 2), jnp.float32)],
)


def _mm2_body(t0_ref, t1_ref, dinv_ref, b1_ref, w2_ref, s0_ref, s1_ref):
  dv = dinv_ref[...]
  t = jnp.concatenate([t0_ref[...], t1_ref[...]], axis=1)
  h = jnp.maximum(t * dv[:, None] + b1_ref[...], 0.0)
  s2 = jnp.dot(h, w2_ref[...], preferred_element_type=jnp.float32)
  s2 = s2 * dv[:, None]
  s0_ref[...] = s2[:, :C // 2]
  s1_ref[...] = s2[:, C // 2:]


_mm2 = pl.pallas_call(
    _mm2_body,
    grid=(GRID,),
    in_specs=[pl.BlockSpec((BLK, H // 2), lambda i: (i, 0)),
              pl.BlockSpec((BLK, H // 2), lambda i: (i, 0)),
              pl.BlockSpec((BLK,), lambda i: (i,)),
              pl.BlockSpec((1, H), lambda i: (0, 0)),
              pl.BlockSpec((H, C), lambda i: (0, 0))],
    out_specs=[pl.BlockSpec((BLK, C // 2), lambda i: (i, 0)),
               pl.BlockSpec((BLK, C // 2), lambda i: (i, 0))],
    out_shape=[jax.ShapeDtypeStruct((NP, C // 2), jnp.float32),
               jax.ShapeDtypeStruct((NP, C // 2), jnp.float32)],
)


def _mm3_body(t0_ref, t1_ref, dinv_ref, b2_ref, o_ref):
  dv = dinv_ref[...]
  t = jnp.concatenate([t0_ref[...], t1_ref[...]], axis=1)
  o_ref[...] = t * dv[:, None] + b2_ref[...]


_mm3 = pl.pallas_call(
    _mm3_body,
    grid=(GRID,),
    in_specs=[pl.BlockSpec((BLK, C // 2), lambda i: (i, 0)),
              pl.BlockSpec((BLK, C // 2), lambda i: (i, 0)),
              pl.BlockSpec((BLK,), lambda i: (i,)),
              pl.BlockSpec((1, C), lambda i: (0, 0))],
    out_specs=pl.BlockSpec((BLK, C), lambda i: (i, 0)),
    out_shape=jax.ShapeDtypeStruct((NP, C), jnp.float32),
)


def kernel(x, adj, W1, b1, W2, b2):
  r = jnp.concatenate([adj[0], jnp.zeros((EP - E,), jnp.int32)])
  c = jnp.concatenate([adj[1], jnp.zeros((EP - E,), jnp.int32)])
  x_pad = jnp.concatenate([x, jnp.zeros((NP - N, F), x.dtype)])

  r_adj, deg_parts = _deg_kernel(r, c)
  cidx_h = c.reshape(NTILE, -1, CH_H)
  ridx_h = r_adj.reshape(NTILE, -1, CH_H)
  cidx_c = c.reshape(NTILE, -1, CH_C)
  ridx_c = r_adj.reshape(NTILE, -1, CH_C)

  dinv2d, s1a, s1b = _mm1(x_pad, W1, deg_parts)

  zh = jnp.zeros((ROWS_PER_TILE, H // 2), jnp.float32)
  t1a, t1b = _spmm_h(s1a, s1b, cidx_h, ridx_h, zh)

  s2a, s2b = _mm2(t1a, t1b, dinv2d, b1.reshape(1, H), W2)

  zc = jnp.zeros((ROWS_PER_TILE, C // 2), jnp.float32)
  t2a, t2b = _spmm_c(s2a, s2b, cidx_c, ridx_c, zc)

  out_pad = _mm3(t2a, t2b, dinv2d, b2.reshape(1, C))
  return out_pad[:N]


# speedup vs baseline: 5.1263x; 1.0286x over previous
import jax
import jax.numpy as jnp
from jax import lax
from jax.experimental import pallas as pl
from jax.experimental.pallas import tpu as pltpu
from jax.experimental.pallas import tpu_sc as plsc

N = 10000
E = 160000
F = 256
H = 256
C = 64

NP = 10240
TRASH = 10000
EP = 163840
NCORE = 2
NTILE = 16
ROWS_PER_TILE = NP // NTILE
A_CHUNK = EP // (NCORE * NTILE)


_mesh = plsc.VectorSubcoreMesh(
    core_axis_name="c", subcore_axis_name="s",
    num_cores=NCORE, num_subcores=NTILE)


def _deg_body(r_in, c_in, r_out, deg_parts, rbuf, cbuf, degbuf):
  cid = lax.axis_index("c")
  sid = lax.axis_index("s")
  wid = sid * NCORE + cid
  off = wid * A_CHUNK
  pltpu.sync_copy(r_in.at[pl.ds(off, A_CHUNK)], rbuf)
  pltpu.sync_copy(c_in.at[pl.ds(off, A_CHUNK)], cbuf)

  zeros16 = jnp.zeros((16,), jnp.float32)
  def zbody(i, carry):
    degbuf[pl.ds(i * 16, 16)] = zeros16
    return carry
  lax.fori_loop(0, NP // 16, zbody, 0)

  ones16 = jnp.ones((16,), jnp.float32)
  def ebody(i, carry):
    rv = rbuf[pl.ds(i * 16, 16)]
    cv = cbuf[pl.ds(i * 16, 16)]
    r2 = jnp.where(rv != cv, rv, TRASH)
    plsc.addupdate_scatter(degbuf, [r2], ones16)
    rbuf[pl.ds(i * 16, 16)] = r2
    return carry
  lax.fori_loop(0, A_CHUNK // 16, ebody, 0)

  pltpu.sync_copy(rbuf, r_out.at[pl.ds(off, A_CHUNK)])
  pltpu.sync_copy(degbuf, deg_parts.at[wid])


_deg_kernel = pl.kernel(
    _deg_body,
    out_type=(jax.ShapeDtypeStruct((EP,), jnp.int32),
              jax.ShapeDtypeStruct((NCORE * NTILE, NP), jnp.float32)),
    mesh=_mesh,
    scratch_types=(pltpu.VMEM((A_CHUNK,), jnp.int32),
                   pltpu.VMEM((A_CHUNK,), jnp.int32),
                   pltpu.VMEM((NP,), jnp.float32)),
    compiler_params=pltpu.CompilerParams(needs_layout_passes=False),
)


def _make_spmm_body(ch, nbuf, nchunks):
  def _spmm_body(t0, t1, cidx, ridx, zrows, o0, o1, cbuf, rbuf, *rest):
    gbufs = rest[:nbuf]
    acc = rest[nbuf]
    sems = rest[nbuf + 1:]
    cid = lax.axis_index("c")
    sid = lax.axis_index("s")
    row0 = sid * ROWS_PER_TILE
    pltpu.sync_copy(zrows, acc.at[pl.ds(row0, ROWS_PER_TILE)])
    pltpu.sync_copy(cidx.at[sid], cbuf)
    pltpu.sync_copy(ridx.at[sid], rbuf)
    plsc.subcore_barrier()

    def run(table):
      for k in range(nbuf):
        pltpu.async_copy(table.at[cbuf.at[k]], gbufs[k], sems[k])
      def step(j, k):
        pltpu.make_async_copy(table.at[cbuf.at[j]], gbufs[k], sems[k]).wait()
        pltpu.sync_copy(gbufs[k], acc.at[rbuf.at[j]], add=True)
        @pl.when(j + nbuf < nchunks)
        def _():
          pltpu.async_copy(table.at[cbuf.at[j + nbuf]], gbufs[k], sems[k])
      def body(i, carry):
        for k in range(nbuf):
          step(i * nbuf + k, k)
        return carry
      lax.fori_loop(0, nchunks // nbuf, body, 0)
      for k in range(nchunks % nbuf):
        step((nchunks // nbuf) * nbuf + k, k)

    @pl.when(cid == 0)
    def _():
      run(t0)
    @pl.when(cid == 1)
    def _():
      run(t1)

    plsc.subcore_barrier()

    @pl.when(cid == 0)
    def _():
      pltpu.sync_copy(acc.at[pl.ds(row0, ROWS_PER_TILE)],
                      o0.at[pl.ds(row0, ROWS_PER_TILE)])
    @pl.when(cid == 1)
    def _():
      pltpu.sync_copy(acc.at[pl.ds(row0, ROWS_PER_TILE)],
                      o1.at[pl.ds(row0, ROWS_PER_TILE)])
  return _spmm_body


def _make_spmm(fw, ch, nbuf):
  nchunks = (EP // NTILE) // ch
  return pl.kernel(
      _make_spmm_body(ch, nbuf, nchunks),
      out_type=(jax.ShapeDtypeStruct((NP, fw), jnp.float32),
                jax.ShapeDtypeStruct((NP, fw), jnp.float32)),
      mesh=_mesh,
      scratch_types=(
          pltpu.VMEM((nchunks, ch), jnp.int32),
          pltpu.VMEM((nchunks, ch), jnp.int32),
          *[pltpu.VMEM((ch, fw), jnp.float32) for _ in range(nbuf)],
          pltpu.VMEM_SHARED((NP, fw), jnp.float32),
          *[pltpu.SemaphoreType.DMA for _ in range(nbuf)],
      ),
      compiler_params=pltpu.CompilerParams(use_tc_tiling_on_sc=False),
  )


CH_H = 64
CH_C = 128
_spmm_h = _make_spmm(H // 2, CH_H, 3)
_spmm_c = _make_spmm(C // 2, CH_C, 4)


BLK = 512
GRID = NP // BLK


def _mm1_body(x_ref, w_ref, deg_ref, dinv_ref, s0_ref, s1_ref):
  deg = jnp.sum(deg_ref[...], axis=0)
  dinv = jnp.where(deg > 0.0, lax.rsqrt(deg), 0.0)
  dinv_ref[...] = dinv
  s = jnp.dot(x_ref[...], w_ref[...], preferred_element_type=jnp.float32)
  s = s * dinv[:, None]
  s0_ref[...] = s[:, :H // 2]
  s1_ref[...] = s[:, H // 2:]


_mm1 = pl.pallas_call(
    _mm1_body,
    grid=(GRID,),
    in_specs=[pl.BlockSpec((BLK, F), lambda i: (i, 0)),
              pl.BlockSpec((F, H), lambda i: (0, 0)),
              pl.BlockSpec((NCORE * NTILE, BLK), lambda i: (0, i))],
    out_specs=[pl.BlockSpec((BLK,), lambda i: (i,)),
               pl.BlockSpec((BLK, H // 2), lambda i: (i, 0)),
               pl.BlockSpec((BLK, H // 2), lambda i: (i, 0))],
    out_shape=[jax.ShapeDtypeStruct((NP,), jnp.float32),
               jax.ShapeDtypeStruct((NP, H // 2), jnp.float32),
               jax.ShapeDtypeStruct((NP, H // 2), jnp.float32)],
)


def _mm2_body(t0_ref, t1_ref, dinv_ref, b1_ref, w2_ref, s0_ref, s1_ref):
  dv = dinv_ref[...]
  t = jnp.concatenate([t0_ref[...], t1_ref[...]], axis=1)
  h = jnp.maximum(t * dv[:, None] + b1_ref[...], 0.0)
  s2 = jnp.dot(h, w2_ref[...], preferred_element_type=jnp.float32)
  s2 = s2 * dv[:, None]
  s0_ref[...] = s2[:, :C // 2]
  s1_ref[...] = s2[:, C // 2:]


_mm2 = pl.pallas_call(
    _mm2_body,
    grid=(GRID,),
    in_specs=[pl.BlockSpec((BLK, H // 2), lambda i: (i, 0)),
              pl.BlockSpec((BLK, H // 2), lambda i: (i, 0)),
              pl.BlockSpec((BLK,), lambda i: (i,)),
              pl.BlockSpec((1, H), lambda i: (0, 0)),
              pl.BlockSpec((H, C), lambda i: (0, 0))],
    out_specs=[pl.BlockSpec((BLK, C // 2), lambda i: (i, 0)),
               pl.BlockSpec((BLK, C // 2), lambda i: (i, 0))],
    out_shape=[jax.ShapeDtypeStruct((NP, C // 2), jnp.float32),
               jax.ShapeDtypeStruct((NP, C // 2), jnp.float32)],
)


def _mm3_body(t0_ref, t1_ref, dinv_ref, b2_ref, o_ref):
  dv = dinv_ref[...]
  t = jnp.concatenate([t0_ref[...], t1_ref[...]], axis=1)
  o_ref[...] = t * dv[:, None] + b2_ref[...]


_mm3 = pl.pallas_call(
    _mm3_body,
    grid=(GRID,),
    in_specs=[pl.BlockSpec((BLK, C // 2), lambda i: (i, 0)),
              pl.BlockSpec((BLK, C // 2), lambda i: (i, 0)),
              pl.BlockSpec((BLK,), lambda i: (i,)),
              pl.BlockSpec((1, C), lambda i: (0, 0))],
    out_specs=pl.BlockSpec((BLK, C), lambda i: (i, 0)),
    out_shape=jax.ShapeDtypeStruct((NP, C), jnp.float32),
)


def kernel(x, adj, W1, b1, W2, b2):
  r = jnp.concatenate([adj[0], jnp.zeros((EP - E,), jnp.int32)])
  c = jnp.concatenate([adj[1], jnp.zeros((EP - E,), jnp.int32)])
  x_pad = jnp.concatenate([x, jnp.zeros((NP - N, F), x.dtype)])

  r_adj, deg_parts = _deg_kernel(r, c)
  cidx_h = c.reshape(NTILE, -1, CH_H)
  ridx_h = r_adj.reshape(NTILE, -1, CH_H)
  cidx_c = c.reshape(NTILE, -1, CH_C)
  ridx_c = r_adj.reshape(NTILE, -1, CH_C)

  dinv2d, s1a, s1b = _mm1(x_pad, W1, deg_parts)

  zh = jnp.zeros((ROWS_PER_TILE, H // 2), jnp.float32)
  t1a, t1b = _spmm_h(s1a, s1b, cidx_h, ridx_h, zh)

  s2a, s2b = _mm2(t1a, t1b, dinv2d, b1.reshape(1, H), W2)

  zc = jnp.zeros((ROWS_PER_TILE, C // 2), jnp.float32)
  t2a, t2b = _spmm_c(s2a, s2b, cidx_c, ridx_c, zc)

  out_pad = _mm3(t2a, t2b, dinv2d, b2.reshape(1, C))
  return out_pad[:N]
